# baseline (speedup 1.0000x reference)
import functools

import jax
import jax.numpy as jnp
from jax.experimental import pallas as pl
from jax.experimental.pallas import tpu as pltpu

N, NB, E, H = 10000, 2500, 160000, 256
NBT = 25


def _mlp_body(zh_ref, wp_ref, bp_ref, m1_ref, b1_ref, m2_ref, b2_ref,
              m3_ref, b3_ref, out_ref):
    zh = zh_ref[...]
    zm = zh @ wp_ref[...] + bp_ref[...]
    z1 = jnp.maximum(zm @ m1_ref[...] + b1_ref[...], 0.0)
    z2 = jnp.maximum(z1 @ m2_ref[...] + b2_ref[...], 0.0)
    out_ref[...] = z2 @ m3_ref[...] + b3_ref[...]


def _mlp_tail(zh, wp, bp, m1, b1, m2, b2, m3, b3):
    return pl.pallas_call(
        _mlp_body,
        out_shape=jax.ShapeDtypeStruct((NB, NBT), jnp.float32),
    )(zh, wp, bp, m1, b1, m2, b2, m3, b3)


def kernel(X, params, A, S, block_ids, edge_index, edge_type):
    p = params
    src, dst = edge_index[0], edge_index[1]

    h = jnp.take(p['atom_tab'], A, axis=0) + jnp.take(
        jnp.take(p['blk_tab'], S, axis=0), block_ids, axis=0)
    h = h @ p['W_in'] + p['b_in']

    dist = jnp.linalg.norm(
        jnp.take(X, src, axis=0) - jnp.take(X, dst, axis=0), axis=-1)

    for l in range(2):
        et = (p['edge_tab'] @ p['We%d' % l])[edge_type]
        m = jax.nn.relu(jnp.take(h, src, axis=0) + et +
                        dist[:, None] * p['wd%d' % l][None, :])
        agg = jax.ops.segment_sum(m, dst, num_segments=N)
        z = agg + h
        z = jax.nn.relu(z @ p['W1_%d' % l] + p['b1_%d' % l])
        h = z @ p['W2_%d' % l] + p['b2_%d' % l]

    cnt = jax.ops.segment_sum(jnp.ones((N,), jnp.float32), block_ids,
                              num_segments=NB)
    Zh = jax.ops.segment_sum(h, block_ids, num_segments=NB) / jnp.clip(
        cnt, 1.0)[:, None]

    wp = p['W_proj'][:H] + p['W_proj'][H:]
    return _mlp_tail(Zh, wp, p['b_proj'], p['M1'], p['m1'], p['M2'], p['m2'],
                     p['M3'], p['m3'])
